# baseline (speedup 1.0000x reference)
import functools

import jax
import jax.numpy as jnp
from jax import lax
from jax.experimental import pallas as pl
from jax.experimental.pallas import tpu as pltpu
from jax.experimental.pallas import tpu_sc as plsc

N = 10000
E = 320000
DF = 128
D = 256
HD = 128
NG = 64
NL = 3

NC, NS = 2, 16
CH = 128
CPT = 160
G = 8
NSUP = CPT // G
EP = NS * CPT * CH
CPW = 80
NPAD = 10240
NPD = 10240
DSTPAD = 10016
RB = 1000

_mesh = plsc.VectorSubcoreMesh(
    core_axis_name="c", subcore_axis_name="s", num_cores=NC, num_subcores=NS)


@functools.partial(
    pl.kernel,
    out_type=(jax.ShapeDtypeStruct((NPAD, HD), jnp.float32),
              jax.ShapeDtypeStruct((NPAD, HD), jnp.float32)),
    mesh=_mesh,
    scratch_types=[
        pltpu.VMEM((2, G, CH), jnp.int32),
        pltpu.VMEM((2, G, CH), jnp.int32),
        pltpu.VMEM((2, CH, HD), jnp.float32),
        pltpu.VMEM_SHARED((NPAD, HD), jnp.float32),
        [pltpu.SemaphoreType.DMA] * 2,
        [pltpu.SemaphoreType.DMA] * 2,
    ],
)
def _edge_scatter(src_hbm, dst_hbm, hw0, hw1, zeros_hbm, out0, out1,
                  src_v, dst_v, rows_v, acc, gsems, isems):
    c = lax.axis_index("c")
    s = lax.axis_index("s")
    pltpu.sync_copy(zeros_hbm, acc.at[pl.ds(s * 640, 640)])
    plsc.subcore_barrier()

    def run(hw, out):
        base = s * CPT

        def idx_load(q, u, issue):
            r = base + u * G
            copy = pltpu.async_copy if issue else pltpu.make_async_copy
            a = copy(src_hbm.at[pl.ds(r, G)], src_v.at[q], isems[q])
            b = copy(dst_hbm.at[pl.ds(r, G)], dst_v.at[q], isems[q])
            if not issue:
                a.wait()
                b.wait()

        pltpu.sync_copy(src_hbm.at[pl.ds(base, G)], src_v.at[0])
        pltpu.sync_copy(dst_hbm.at[pl.ds(base, G)], dst_v.at[0])
        idx_load(1, 1, True)
        pltpu.async_copy(hw.at[src_v.at[0, 0]], rows_v.at[0], gsems[0])

        def step(up, carry):
            for pu in range(2):
                u = 2 * up + pu
                q = 1 - pu
                for ch in range(G):
                    b = ch % 2
                    nb = 1 - b
                    pltpu.make_async_copy(hw.at[src_v.at[pu, ch]],
                                          rows_v.at[b], gsems[b]).wait()
                    if ch < G - 1:
                        pltpu.async_copy(hw.at[src_v.at[pu, ch + 1]],
                                         rows_v.at[nb], gsems[nb])
                    else:
                        @pl.when(u + 1 < NSUP)
                        def _():
                            idx_load(q, u + 1, False)
                            pltpu.async_copy(hw.at[src_v.at[q, 0]],
                                             rows_v.at[nb], gsems[nb])
                    pltpu.sync_copy(rows_v.at[b], acc.at[dst_v.at[pu, ch]],
                                    add=True)

                @pl.when(u + 2 < NSUP)
                def _():
                    idx_load(pu, u + 2, True)
            return carry

        lax.fori_loop(0, NSUP // 2, step, 0)
        plsc.subcore_barrier()
        pltpu.sync_copy(acc.at[pl.ds(s * 640, 640)],
                        out.at[pl.ds(s * 640, 640)])

    @pl.when(c == 0)
    def _():
        run(hw0, out0)

    @pl.when(c == 1)
    def _():
        run(hw1, out1)


@functools.partial(
    pl.kernel,
    out_type=(jax.ShapeDtypeStruct((NPD,), jnp.float32),
              jax.ShapeDtypeStruct((NPD,), jnp.float32)),
    mesh=_mesh,
    scratch_types=[
        pltpu.VMEM((2, CH), jnp.int32),
        pltpu.VMEM((CH,), jnp.float32),
        pltpu.VMEM_SHARED((NPD,), jnp.float32),
        pltpu.SemaphoreType.DMA,
        pltpu.SemaphoreType.DMA,
    ],
)
def _degree(dst_hbm, ones_hbm, zeros_hbm, out0, out1,
            dst_v, ones_v, acc, sem0, sem1):
    c = lax.axis_index("c")
    s = lax.axis_index("s")
    pltpu.sync_copy(zeros_hbm, acc.at[pl.ds(s * 640, 640)])
    pltpu.sync_copy(ones_hbm, ones_v)
    plsc.subcore_barrier()

    sems = (sem0, sem1)
    base = (s * NC + c) * CPW
    for b in range(2):
        pltpu.sync_copy(dst_hbm.at[pl.ds(base + b, 1)], dst_v.at[pl.ds(b, 1)])
        pltpu.async_copy(ones_v, acc.at[dst_v.at[b]], sems[b], add=True)

    def step(g, carry):
        for b in range(2):
            j = 2 * g + b
            pltpu.make_async_copy(ones_v, acc.at[dst_v.at[b]],
                                  sems[b]).wait()

            @pl.when(j + 2 < CPW)
            def _():
                pltpu.sync_copy(dst_hbm.at[pl.ds(base + j + 2, 1)],
                                dst_v.at[pl.ds(b, 1)])
                pltpu.async_copy(ones_v, acc.at[dst_v.at[b]], sems[b],
                                 add=True)
        return carry

    lax.fori_loop(0, CPW // 2, step, 0)
    plsc.subcore_barrier()

    @pl.when(c == 0)
    def _():
        pltpu.sync_copy(acc.at[pl.ds(s * 640, 640)],
                        out0.at[pl.ds(s * 640, 640)])

    @pl.when(c == 1)
    def _():
        pltpu.sync_copy(acc.at[pl.ds(s * 640, 640)],
                        out1.at[pl.ds(s * 640, 640)])


def _embed_body(x_ref, w_ref, b_ref, d0_ref, d1_ref, h_ref, dinv_ref):
    deg = d0_ref[...] + d1_ref[...] + 1.0
    dinv_ref[...] = lax.rsqrt(deg)
    h_ref[...] = jnp.dot(x_ref[...], w_ref[...],
                         preferred_element_type=jnp.float32) + b_ref[...]


_embed = pl.pallas_call(
    _embed_body,
    grid=(N // RB,),
    in_specs=[
        pl.BlockSpec((RB, DF), lambda i: (i, 0)),
        pl.BlockSpec((DF, D), lambda i: (0, 0)),
        pl.BlockSpec((1, D), lambda i: (0, 0)),
        pl.BlockSpec((RB, 1), lambda i: (i, 0)),
        pl.BlockSpec((RB, 1), lambda i: (i, 0)),
    ],
    out_specs=[
        pl.BlockSpec((RB, D), lambda i: (i, 0)),
        pl.BlockSpec((RB, 1), lambda i: (i, 0)),
    ],
    out_shape=[
        jax.ShapeDtypeStruct((N, D), jnp.float32),
        jax.ShapeDtypeStruct((N, 1), jnp.float32),
    ],
)


def _mm_scale_body(h_ref, w_ref, dinv_ref, o0_ref, o1_ref):
    hw = jnp.dot(h_ref[...], w_ref[...],
                 preferred_element_type=jnp.float32) * dinv_ref[...]
    o0_ref[...] = hw[:, :HD]
    o1_ref[...] = hw[:, HD:]


_mm_scale = pl.pallas_call(
    _mm_scale_body,
    grid=(N // RB,),
    in_specs=[
        pl.BlockSpec((RB, D), lambda i: (i, 0)),
        pl.BlockSpec((D, D), lambda i: (0, 0)),
        pl.BlockSpec((RB, 1), lambda i: (i, 0)),
    ],
    out_specs=[
        pl.BlockSpec((RB, HD), lambda i: (i, 0)),
        pl.BlockSpec((RB, HD), lambda i: (i, 0)),
    ],
    out_shape=[
        jax.ShapeDtypeStruct((N, HD), jnp.float32),
        jax.ShapeDtypeStruct((N, HD), jnp.float32),
    ],
)


def _stats_body(a0_ref, a1_ref, hw0_ref, hw1_ref, dinv_ref, bc_ref,
                t_ref, s1_ref, s2_ref):
    dinv = dinv_ref[...]
    t0 = dinv * (a0_ref[...] + hw0_ref[...])
    t1 = dinv * (a1_ref[...] + hw1_ref[...])
    t = jnp.concatenate([t0, t1], axis=1) + bc_ref[...]
    t_ref[...] = t

    @pl.when(pl.program_id(0) == 0)
    def _():
        s1_ref[...] = jnp.zeros_like(s1_ref)
        s2_ref[...] = jnp.zeros_like(s2_ref)

    s1_ref[...] += jnp.sum(t, axis=0, keepdims=True)
    s2_ref[...] += jnp.sum(t * t, axis=0, keepdims=True)


_stats = pl.pallas_call(
    _stats_body,
    grid=(N // RB,),
    in_specs=[
        pl.BlockSpec((RB, HD), lambda i: (i, 0)),
        pl.BlockSpec((RB, HD), lambda i: (i, 0)),
        pl.BlockSpec((RB, HD), lambda i: (i, 0)),
        pl.BlockSpec((RB, HD), lambda i: (i, 0)),
        pl.BlockSpec((RB, 1), lambda i: (i, 0)),
        pl.BlockSpec((1, D), lambda i: (0, 0)),
    ],
    out_specs=[
        pl.BlockSpec((RB, D), lambda i: (i, 0)),
        pl.BlockSpec((1, D), lambda i: (0, 0)),
        pl.BlockSpec((1, D), lambda i: (0, 0)),
    ],
    out_shape=[
        jax.ShapeDtypeStruct((N, D), jnp.float32),
        jax.ShapeDtypeStruct((1, D), jnp.float32),
        jax.ShapeDtypeStruct((1, D), jnp.float32),
    ],
)


def _bn_body(t_ref, s1_ref, s2_ref, g_ref, be_ref, hin_ref, ho_ref):
    inv_n = 1.0 / N
    mu = s1_ref[...] * inv_n
    var = s2_ref[...] * inv_n - mu * mu
    xn = (t_ref[...] - mu) * lax.rsqrt(var + 1e-5) * g_ref[...] + be_ref[...]
    ho_ref[...] = hin_ref[...] + jnp.maximum(xn, 0.0)


_bn_relu_res = pl.pallas_call(
    _bn_body,
    grid=(N // RB,),
    in_specs=[
        pl.BlockSpec((RB, D), lambda i: (i, 0)),
        pl.BlockSpec((1, D), lambda i: (0, 0)),
        pl.BlockSpec((1, D), lambda i: (0, 0)),
        pl.BlockSpec((1, D), lambda i: (0, 0)),
        pl.BlockSpec((1, D), lambda i: (0, 0)),
        pl.BlockSpec((RB, D), lambda i: (i, 0)),
    ],
    out_specs=pl.BlockSpec((RB, D), lambda i: (i, 0)),
    out_shape=jax.ShapeDtypeStruct((N, D), jnp.float32),
)


def _pool_body(h_ref, b_ref, w1_ref, b1_ref, w2_ref, b2_ref, w3_ref, b3_ref,
               out_ref, g_acc, c_acc):
    i = pl.program_id(0)

    @pl.when(i == 0)
    def _():
        g_acc[...] = jnp.zeros_like(g_acc)
        c_acc[...] = jnp.zeros_like(c_acc)

    gid = lax.broadcasted_iota(jnp.int32, (RB, NG), 1)
    onehot = jnp.where(b_ref[...] == gid, 1.0, 0.0).astype(jnp.float32)
    g_acc[...] += lax.dot_general(onehot, h_ref[...], (((0,), (0,)), ((), ())),
                                  preferred_element_type=jnp.float32)
    c_acc[...] += lax.dot_general(onehot, jnp.ones((RB, 1), jnp.float32),
                                  (((0,), (0,)), ((), ())),
                                  preferred_element_type=jnp.float32)

    @pl.when(i == pl.num_programs(0) - 1)
    def _():
        g = g_acc[...] / jnp.maximum(c_acc[...], 1.0)
        g = jnp.maximum(jnp.dot(g, w1_ref[...],
                                preferred_element_type=jnp.float32)
                        + b1_ref[...], 0.0)
        g = jnp.maximum(jnp.dot(g, w2_ref[...],
                                preferred_element_type=jnp.float32)
                        + b2_ref[...], 0.0)
        out_ref[...] = jnp.dot(g, w3_ref[...],
                               preferred_element_type=jnp.float32) + b3_ref[...]


_pool_mlp = pl.pallas_call(
    _pool_body,
    grid=(N // RB,),
    in_specs=[
        pl.BlockSpec((RB, D), lambda i: (i, 0)),
        pl.BlockSpec((RB, 1), lambda i: (i, 0)),
        pl.BlockSpec((D, D // 2), lambda i: (0, 0)),
        pl.BlockSpec((1, D // 2), lambda i: (0, 0)),
        pl.BlockSpec((D // 2, D // 4), lambda i: (0, 0)),
        pl.BlockSpec((1, D // 4), lambda i: (0, 0)),
        pl.BlockSpec((D // 4, 10), lambda i: (0, 0)),
        pl.BlockSpec((1, 10), lambda i: (0, 0)),
    ],
    out_specs=pl.BlockSpec((NG, 10), lambda i: (0, 0)),
    out_shape=jax.ShapeDtypeStruct((NG, 10), jnp.float32),
    scratch_shapes=[
        pltpu.VMEM((NG, D), jnp.float32),
        pltpu.VMEM((NG, 1), jnp.float32),
    ],
)


def kernel(x, edge_index, batch, W0, b0, Wc, bc, gamma, beta,
           W1, b1, W2, b2, W3, b3):
    src = edge_index[0]
    dst = edge_index[1]
    src2d = jnp.concatenate(
        [src, jnp.zeros((EP - E,), jnp.int32)]).reshape(NS * CPT, CH)
    dst2d = jnp.concatenate(
        [dst, jnp.full((EP - E,), DSTPAD, jnp.int32)]).reshape(NS * CPT, CH)

    zeros2d = jnp.zeros((640, HD), jnp.float32)
    zeros1d = jnp.zeros((640,), jnp.float32)
    ones1d = jnp.ones((CH,), jnp.float32)

    d0p, d1p = _degree(dst2d, ones1d, zeros1d)
    d0 = d0p[:N].reshape(N, 1)
    d1 = d1p[:N].reshape(N, 1)

    h, dinv = _embed(x, W0, b0.reshape(1, D), d0, d1)
    for i in range(NL):
        hw0, hw1 = _mm_scale(h, Wc[i], dinv)
        a0, a1 = _edge_scatter(src2d, dst2d, hw0, hw1, zeros2d)
        t, s1, s2 = _stats(a0, a1, hw0, hw1, dinv, bc[i].reshape(1, D))
        h = _bn_relu_res(t, s1, s2, gamma[i].reshape(1, D),
                         beta[i].reshape(1, D), h)

    out = _pool_mlp(h, batch.reshape(N, 1), W1, b1.reshape(1, D // 2),
                    W2, b2.reshape(1, D // 4), W3, b3.reshape(1, 10))
    return out


# speedup vs baseline: 1.0709x; 1.0709x over previous
import functools

import jax
import jax.numpy as jnp
from jax import lax
from jax.experimental import pallas as pl
from jax.experimental.pallas import tpu as pltpu
from jax.experimental.pallas import tpu_sc as plsc

N = 10000
E = 320000
DF = 128
D = 256
HD = 128
NG = 64
NL = 3

NC, NS = 2, 16
CH = 128
CPT = 160
G = 8
NSUP = CPT // G
EP = NS * CPT * CH
CPW = 80
NPAD = 10240
NPD = 10240
DSTPAD = 10016
RB = 1000

_mesh = plsc.VectorSubcoreMesh(
    core_axis_name="c", subcore_axis_name="s", num_cores=NC, num_subcores=NS)


@functools.partial(
    pl.kernel,
    out_type=(jax.ShapeDtypeStruct((NPAD, HD), jnp.float32),
              jax.ShapeDtypeStruct((NPAD, HD), jnp.float32)),
    mesh=_mesh,
    scratch_types=[
        pltpu.VMEM((2, G, CH), jnp.int32),
        pltpu.VMEM((2, G, CH), jnp.int32),
        pltpu.VMEM((2, CH, HD), jnp.float32),
        pltpu.VMEM_SHARED((NPAD, HD), jnp.float32),
        [pltpu.SemaphoreType.DMA] * 2,
        [pltpu.SemaphoreType.DMA] * 2,
    ],
)
def _edge_scatter(src_hbm, dst_hbm, hw0, hw1, zeros_hbm, out0, out1,
                  src_v, dst_v, rows_v, acc, gsems, isems):
    c = lax.axis_index("c")
    s = lax.axis_index("s")
    pltpu.sync_copy(zeros_hbm, acc.at[pl.ds(s * 640, 640)])
    plsc.subcore_barrier()

    def run(hw, out):
        base = s * CPT

        def idx_load(q, u, issue):
            r = base + u * G
            copy = pltpu.async_copy if issue else pltpu.make_async_copy
            a = copy(src_hbm.at[pl.ds(r, G)], src_v.at[q], isems[q])
            b = copy(dst_hbm.at[pl.ds(r, G)], dst_v.at[q], isems[q])
            if not issue:
                a.wait()
                b.wait()

        pltpu.sync_copy(src_hbm.at[pl.ds(base, G)], src_v.at[0])
        pltpu.sync_copy(dst_hbm.at[pl.ds(base, G)], dst_v.at[0])
        idx_load(1, 1, True)
        for b0 in range(2):
            pltpu.async_copy(hw.at[src_v.at[0, b0]], rows_v.at[b0],
                             gsems[b0])

        def step(up, carry):
            for pu in range(2):
                u = 2 * up + pu
                q = 1 - pu
                for ch in range(G):
                    b = ch % 2
                    pltpu.make_async_copy(hw.at[src_v.at[pu, ch]],
                                          rows_v.at[b], gsems[b]).wait()
                    pltpu.sync_copy(rows_v.at[b], acc.at[dst_v.at[pu, ch]],
                                    add=True)
                    if ch < G - 2:
                        pltpu.async_copy(hw.at[src_v.at[pu, ch + 2]],
                                         rows_v.at[b], gsems[b])
                    else:
                        @pl.when(u + 1 < NSUP)
                        def _():
                            if ch == G - 2:
                                idx_load(q, u + 1, False)
                            pltpu.async_copy(
                                hw.at[src_v.at[q, ch + 2 - G]],
                                rows_v.at[b], gsems[b])

                @pl.when(u + 2 < NSUP)
                def _():
                    idx_load(pu, u + 2, True)
            return carry

        lax.fori_loop(0, NSUP // 2, step, 0)
        plsc.subcore_barrier()
        pltpu.sync_copy(acc.at[pl.ds(s * 640, 640)],
                        out.at[pl.ds(s * 640, 640)])

    @pl.when(c == 0)
    def _():
        run(hw0, out0)

    @pl.when(c == 1)
    def _():
        run(hw1, out1)


@functools.partial(
    pl.kernel,
    out_type=(jax.ShapeDtypeStruct((NPD,), jnp.float32),
              jax.ShapeDtypeStruct((NPD,), jnp.float32)),
    mesh=_mesh,
    scratch_types=[
        pltpu.VMEM((2, CH), jnp.int32),
        pltpu.VMEM((CH,), jnp.float32),
        pltpu.VMEM_SHARED((NPD,), jnp.float32),
        pltpu.SemaphoreType.DMA,
        pltpu.SemaphoreType.DMA,
    ],
)
def _degree(dst_hbm, ones_hbm, zeros_hbm, out0, out1,
            dst_v, ones_v, acc, sem0, sem1):
    c = lax.axis_index("c")
    s = lax.axis_index("s")
    pltpu.sync_copy(zeros_hbm, acc.at[pl.ds(s * 640, 640)])
    pltpu.sync_copy(ones_hbm, ones_v)
    plsc.subcore_barrier()

    sems = (sem0, sem1)
    base = (s * NC + c) * CPW
    for b in range(2):
        pltpu.sync_copy(dst_hbm.at[pl.ds(base + b, 1)], dst_v.at[pl.ds(b, 1)])
        pltpu.async_copy(ones_v, acc.at[dst_v.at[b]], sems[b], add=True)

    def step(g, carry):
        for b in range(2):
            j = 2 * g + b
            pltpu.make_async_copy(ones_v, acc.at[dst_v.at[b]],
                                  sems[b]).wait()

            @pl.when(j + 2 < CPW)
            def _():
                pltpu.sync_copy(dst_hbm.at[pl.ds(base + j + 2, 1)],
                                dst_v.at[pl.ds(b, 1)])
                pltpu.async_copy(ones_v, acc.at[dst_v.at[b]], sems[b],
                                 add=True)
        return carry

    lax.fori_loop(0, CPW // 2, step, 0)
    plsc.subcore_barrier()

    @pl.when(c == 0)
    def _():
        pltpu.sync_copy(acc.at[pl.ds(s * 640, 640)],
                        out0.at[pl.ds(s * 640, 640)])

    @pl.when(c == 1)
    def _():
        pltpu.sync_copy(acc.at[pl.ds(s * 640, 640)],
                        out1.at[pl.ds(s * 640, 640)])


def _embed_body(x_ref, w_ref, b_ref, d0_ref, d1_ref, h_ref, dinv_ref):
    deg = d0_ref[...] + d1_ref[...] + 1.0
    dinv_ref[...] = lax.rsqrt(deg)
    h_ref[...] = jnp.dot(x_ref[...], w_ref[...],
                         preferred_element_type=jnp.float32) + b_ref[...]


_embed = pl.pallas_call(
    _embed_body,
    grid=(N // RB,),
    in_specs=[
        pl.BlockSpec((RB, DF), lambda i: (i, 0)),
        pl.BlockSpec((DF, D), lambda i: (0, 0)),
        pl.BlockSpec((1, D), lambda i: (0, 0)),
        pl.BlockSpec((RB, 1), lambda i: (i, 0)),
        pl.BlockSpec((RB, 1), lambda i: (i, 0)),
    ],
    out_specs=[
        pl.BlockSpec((RB, D), lambda i: (i, 0)),
        pl.BlockSpec((RB, 1), lambda i: (i, 0)),
    ],
    out_shape=[
        jax.ShapeDtypeStruct((N, D), jnp.float32),
        jax.ShapeDtypeStruct((N, 1), jnp.float32),
    ],
)


def _mm_scale_body(h_ref, w_ref, dinv_ref, o0_ref, o1_ref):
    hw = jnp.dot(h_ref[...], w_ref[...],
                 preferred_element_type=jnp.float32) * dinv_ref[...]
    o0_ref[...] = hw[:, :HD]
    o1_ref[...] = hw[:, HD:]


_mm_scale = pl.pallas_call(
    _mm_scale_body,
    grid=(N // RB,),
    in_specs=[
        pl.BlockSpec((RB, D), lambda i: (i, 0)),
        pl.BlockSpec((D, D), lambda i: (0, 0)),
        pl.BlockSpec((RB, 1), lambda i: (i, 0)),
    ],
    out_specs=[
        pl.BlockSpec((RB, HD), lambda i: (i, 0)),
        pl.BlockSpec((RB, HD), lambda i: (i, 0)),
    ],
    out_shape=[
        jax.ShapeDtypeStruct((N, HD), jnp.float32),
        jax.ShapeDtypeStruct((N, HD), jnp.float32),
    ],
)


def _stats_body(a0_ref, a1_ref, hw0_ref, hw1_ref, dinv_ref, bc_ref,
                t_ref, s1_ref, s2_ref):
    dinv = dinv_ref[...]
    t0 = dinv * (a0_ref[...] + hw0_ref[...])
    t1 = dinv * (a1_ref[...] + hw1_ref[...])
    t = jnp.concatenate([t0, t1], axis=1) + bc_ref[...]
    t_ref[...] = t

    @pl.when(pl.program_id(0) == 0)
    def _():
        s1_ref[...] = jnp.zeros_like(s1_ref)
        s2_ref[...] = jnp.zeros_like(s2_ref)

    s1_ref[...] += jnp.sum(t, axis=0, keepdims=True)
    s2_ref[...] += jnp.sum(t * t, axis=0, keepdims=True)


_stats = pl.pallas_call(
    _stats_body,
    grid=(N // RB,),
    in_specs=[
        pl.BlockSpec((RB, HD), lambda i: (i, 0)),
        pl.BlockSpec((RB, HD), lambda i: (i, 0)),
        pl.BlockSpec((RB, HD), lambda i: (i, 0)),
        pl.BlockSpec((RB, HD), lambda i: (i, 0)),
        pl.BlockSpec((RB, 1), lambda i: (i, 0)),
        pl.BlockSpec((1, D), lambda i: (0, 0)),
    ],
    out_specs=[
        pl.BlockSpec((RB, D), lambda i: (i, 0)),
        pl.BlockSpec((1, D), lambda i: (0, 0)),
        pl.BlockSpec((1, D), lambda i: (0, 0)),
    ],
    out_shape=[
        jax.ShapeDtypeStruct((N, D), jnp.float32),
        jax.ShapeDtypeStruct((1, D), jnp.float32),
        jax.ShapeDtypeStruct((1, D), jnp.float32),
    ],
)


def _bn_body(t_ref, s1_ref, s2_ref, g_ref, be_ref, hin_ref, ho_ref):
    inv_n = 1.0 / N
    mu = s1_ref[...] * inv_n
    var = s2_ref[...] * inv_n - mu * mu
    xn = (t_ref[...] - mu) * lax.rsqrt(var + 1e-5) * g_ref[...] + be_ref[...]
    ho_ref[...] = hin_ref[...] + jnp.maximum(xn, 0.0)


_bn_relu_res = pl.pallas_call(
    _bn_body,
    grid=(N // RB,),
    in_specs=[
        pl.BlockSpec((RB, D), lambda i: (i, 0)),
        pl.BlockSpec((1, D), lambda i: (0, 0)),
        pl.BlockSpec((1, D), lambda i: (0, 0)),
        pl.BlockSpec((1, D), lambda i: (0, 0)),
        pl.BlockSpec((1, D), lambda i: (0, 0)),
        pl.BlockSpec((RB, D), lambda i: (i, 0)),
    ],
    out_specs=pl.BlockSpec((RB, D), lambda i: (i, 0)),
    out_shape=jax.ShapeDtypeStruct((N, D), jnp.float32),
)


def _pool_body(h_ref, b_ref, w1_ref, b1_ref, w2_ref, b2_ref, w3_ref, b3_ref,
               out_ref, g_acc, c_acc):
    i = pl.program_id(0)

    @pl.when(i == 0)
    def _():
        g_acc[...] = jnp.zeros_like(g_acc)
        c_acc[...] = jnp.zeros_like(c_acc)

    gid = lax.broadcasted_iota(jnp.int32, (RB, NG), 1)
    onehot = jnp.where(b_ref[...] == gid, 1.0, 0.0).astype(jnp.float32)
    g_acc[...] += lax.dot_general(onehot, h_ref[...], (((0,), (0,)), ((), ())),
                                  preferred_element_type=jnp.float32)
    c_acc[...] += lax.dot_general(onehot, jnp.ones((RB, 1), jnp.float32),
                                  (((0,), (0,)), ((), ())),
                                  preferred_element_type=jnp.float32)

    @pl.when(i == pl.num_programs(0) - 1)
    def _():
        g = g_acc[...] / jnp.maximum(c_acc[...], 1.0)
        g = jnp.maximum(jnp.dot(g, w1_ref[...],
                                preferred_element_type=jnp.float32)
                        + b1_ref[...], 0.0)
        g = jnp.maximum(jnp.dot(g, w2_ref[...],
                                preferred_element_type=jnp.float32)
                        + b2_ref[...], 0.0)
        out_ref[...] = jnp.dot(g, w3_ref[...],
                               preferred_element_type=jnp.float32) + b3_ref[...]


_pool_mlp = pl.pallas_call(
    _pool_body,
    grid=(N // RB,),
    in_specs=[
        pl.BlockSpec((RB, D), lambda i: (i, 0)),
        pl.BlockSpec((RB, 1), lambda i: (i, 0)),
        pl.BlockSpec((D, D // 2), lambda i: (0, 0)),
        pl.BlockSpec((1, D // 2), lambda i: (0, 0)),
        pl.BlockSpec((D // 2, D // 4), lambda i: (0, 0)),
        pl.BlockSpec((1, D // 4), lambda i: (0, 0)),
        pl.BlockSpec((D // 4, 10), lambda i: (0, 0)),
        pl.BlockSpec((1, 10), lambda i: (0, 0)),
    ],
    out_specs=pl.BlockSpec((NG, 10), lambda i: (0, 0)),
    out_shape=jax.ShapeDtypeStruct((NG, 10), jnp.float32),
    scratch_shapes=[
        pltpu.VMEM((NG, D), jnp.float32),
        pltpu.VMEM((NG, 1), jnp.float32),
    ],
)


def kernel(x, edge_index, batch, W0, b0, Wc, bc, gamma, beta,
           W1, b1, W2, b2, W3, b3):
    src = edge_index[0]
    dst = edge_index[1]
    src2d = jnp.concatenate(
        [src, jnp.zeros((EP - E,), jnp.int32)]).reshape(NS * CPT, CH)
    dst2d = jnp.concatenate(
        [dst, jnp.full((EP - E,), DSTPAD, jnp.int32)]).reshape(NS * CPT, CH)

    zeros2d = jnp.zeros((640, HD), jnp.float32)
    zeros1d = jnp.zeros((640,), jnp.float32)
    ones1d = jnp.ones((CH,), jnp.float32)

    d0p, d1p = _degree(dst2d, ones1d, zeros1d)
    d0 = d0p[:N].reshape(N, 1)
    d1 = d1p[:N].reshape(N, 1)

    h, dinv = _embed(x, W0, b0.reshape(1, D), d0, d1)
    for i in range(NL):
        hw0, hw1 = _mm_scale(h, Wc[i], dinv)
        a0, a1 = _edge_scatter(src2d, dst2d, hw0, hw1, zeros2d)
        t, s1, s2 = _stats(a0, a1, hw0, hw1, dinv, bc[i].reshape(1, D))
        h = _bn_relu_res(t, s1, s2, gamma[i].reshape(1, D),
                         beta[i].reshape(1, D), h)

    out = _pool_mlp(h, batch.reshape(N, 1), W1, b1.reshape(1, D // 2),
                    W2, b2.reshape(1, D // 4), W3, b3.reshape(1, 10))
    return out


# speedup vs baseline: 1.2335x; 1.1517x over previous
import functools

import jax
import jax.numpy as jnp
from jax import lax
from jax.experimental import pallas as pl
from jax.experimental.pallas import tpu as pltpu
from jax.experimental.pallas import tpu_sc as plsc

N = 10000
E = 320000
DF = 128
D = 256
HD = 128
NG = 64
NL = 3

NC, NS = 2, 16
CH = 128
CPT = 158
EP = NS * CPT * CH
CPW = 80
EPD = NC * NS * CPW * CH
NPAD = 10240
DSTPAD = 10016
RB = 1000

_mesh = plsc.VectorSubcoreMesh(
    core_axis_name="c", subcore_axis_name="s", num_cores=NC, num_subcores=NS)


@functools.partial(
    pl.kernel,
    out_type=(jax.ShapeDtypeStruct((NPAD, HD), jnp.float32),
              jax.ShapeDtypeStruct((NPAD, HD), jnp.float32)),
    mesh=_mesh,
    scratch_types=[
        pltpu.VMEM((2, CH), jnp.int32),
        pltpu.VMEM((2, CH), jnp.int32),
        pltpu.VMEM((CH, HD), jnp.float32),
        pltpu.VMEM((CH, HD), jnp.float32),
        pltpu.VMEM_SHARED((NPAD, HD), jnp.float32),
        pltpu.SemaphoreType.DMA,
        pltpu.SemaphoreType.DMA,
    ],
)
def _edge_scatter(src_hbm, dst_hbm, hw0, hw1, zeros_hbm, out0, out1,
                  src_v, dst_v, rows0, rows1, acc, sem0, sem1):
    c = lax.axis_index("c")
    s = lax.axis_index("s")
    pltpu.sync_copy(zeros_hbm, acc.at[pl.ds(s * 640, 640)])
    plsc.subcore_barrier()

    def run(hw, out):
        rows = (rows0, rows1)
        sems = (sem0, sem1)
        base = s * CPT

        for b in range(2):
            pltpu.sync_copy(src_hbm.at[pl.ds(base + b, 1)],
                            src_v.at[pl.ds(b, 1)])
            pltpu.sync_copy(dst_hbm.at[pl.ds(base + b, 1)],
                            dst_v.at[pl.ds(b, 1)])
            pltpu.async_copy(hw.at[src_v.at[b]], rows[b], sems[b])

        def step(g, carry):
            for b in range(2):
                j = 2 * g + b
                pltpu.make_async_copy(hw.at[src_v.at[b]], rows[b],
                                      sems[b]).wait()
                pltpu.sync_copy(rows[b], acc.at[dst_v.at[b]], add=True)

                @pl.when(j + 2 < CPT)
                def _():
                    r = base + j + 2
                    pltpu.sync_copy(src_hbm.at[pl.ds(r, 1)],
                                    src_v.at[pl.ds(b, 1)])
                    pltpu.sync_copy(dst_hbm.at[pl.ds(r, 1)],
                                    dst_v.at[pl.ds(b, 1)])
                    pltpu.async_copy(hw.at[src_v.at[b]], rows[b], sems[b])
            return carry

        lax.fori_loop(0, CPT // 2, step, 0)
        plsc.subcore_barrier()
        pltpu.sync_copy(acc.at[pl.ds(s * 640, 640)],
                        out.at[pl.ds(s * 640, 640)])

    @pl.when(c == 0)
    def _():
        run(hw0, out0)

    @pl.when(c == 1)
    def _():
        run(hw1, out1)


@functools.partial(
    pl.kernel,
    out_type=(jax.ShapeDtypeStruct((NPAD,), jnp.float32),
              jax.ShapeDtypeStruct((NPAD,), jnp.float32)),
    mesh=_mesh,
    scratch_types=[
        pltpu.VMEM((2, CH), jnp.int32),
        pltpu.VMEM((CH,), jnp.float32),
        pltpu.VMEM_SHARED((NPAD,), jnp.float32),
        pltpu.SemaphoreType.DMA,
        pltpu.SemaphoreType.DMA,
    ],
)
def _degree(dst_hbm, ones_hbm, zeros_hbm, out0, out1,
            dst_v, ones_v, acc, sem0, sem1):
    c = lax.axis_index("c")
    s = lax.axis_index("s")
    pltpu.sync_copy(zeros_hbm, acc.at[pl.ds(s * 640, 640)])
    pltpu.sync_copy(ones_hbm, ones_v)
    plsc.subcore_barrier()

    sems = (sem0, sem1)
    base = (s * NC + c) * CPW
    for b in range(2):
        pltpu.sync_copy(dst_hbm.at[pl.ds(base + b, 1)], dst_v.at[pl.ds(b, 1)])
        pltpu.async_copy(ones_v, acc.at[dst_v.at[b]], sems[b], add=True)

    def step(g, carry):
        for b in range(2):
            j = 2 * g + b
            pltpu.make_async_copy(ones_v, acc.at[dst_v.at[b]],
                                  sems[b]).wait()

            @pl.when(j + 2 < CPW)
            def _():
                pltpu.sync_copy(dst_hbm.at[pl.ds(base + j + 2, 1)],
                                dst_v.at[pl.ds(b, 1)])
                pltpu.async_copy(ones_v, acc.at[dst_v.at[b]], sems[b],
                                 add=True)
        return carry

    lax.fori_loop(0, CPW // 2, step, 0)
    plsc.subcore_barrier()

    @pl.when(c == 0)
    def _():
        pltpu.sync_copy(acc.at[pl.ds(s * 640, 640)],
                        out0.at[pl.ds(s * 640, 640)])

    @pl.when(c == 1)
    def _():
        pltpu.sync_copy(acc.at[pl.ds(s * 640, 640)],
                        out1.at[pl.ds(s * 640, 640)])


def _embed_body(x_ref, w_ref, b_ref, d0_ref, d1_ref, h_ref, dinv_ref):
    deg = d0_ref[...] + d1_ref[...] + 1.0
    dinv_ref[...] = lax.rsqrt(deg)
    h_ref[...] = jnp.dot(x_ref[...], w_ref[...],
                         preferred_element_type=jnp.float32) + b_ref[...]


_embed = pl.pallas_call(
    _embed_body,
    grid=(N // RB,),
    in_specs=[
        pl.BlockSpec((RB, DF), lambda i: (i, 0)),
        pl.BlockSpec((DF, D), lambda i: (0, 0)),
        pl.BlockSpec((1, D), lambda i: (0, 0)),
        pl.BlockSpec((RB, 1), lambda i: (i, 0)),
        pl.BlockSpec((RB, 1), lambda i: (i, 0)),
    ],
    out_specs=[
        pl.BlockSpec((RB, D), lambda i: (i, 0)),
        pl.BlockSpec((RB, 1), lambda i: (i, 0)),
    ],
    out_shape=[
        jax.ShapeDtypeStruct((N, D), jnp.float32),
        jax.ShapeDtypeStruct((N, 1), jnp.float32),
    ],
)


def _mm_scale_body(h_ref, w_ref, dinv_ref, o0_ref, o1_ref):
    hw = jnp.dot(h_ref[...], w_ref[...],
                 preferred_element_type=jnp.float32) * dinv_ref[...]
    o0_ref[...] = hw[:, :HD]
    o1_ref[...] = hw[:, HD:]


_mm_scale = pl.pallas_call(
    _mm_scale_body,
    grid=(N // RB,),
    in_specs=[
        pl.BlockSpec((RB, D), lambda i: (i, 0)),
        pl.BlockSpec((D, D), lambda i: (0, 0)),
        pl.BlockSpec((RB, 1), lambda i: (i, 0)),
    ],
    out_specs=[
        pl.BlockSpec((RB, HD), lambda i: (i, 0)),
        pl.BlockSpec((RB, HD), lambda i: (i, 0)),
    ],
    out_shape=[
        jax.ShapeDtypeStruct((N, HD), jnp.float32),
        jax.ShapeDtypeStruct((N, HD), jnp.float32),
    ],
)


def _stats_body(a0_ref, a1_ref, hw0_ref, hw1_ref, dinv_ref, bc_ref,
                t_ref, s1_ref, s2_ref):
    dinv = dinv_ref[...]
    t0 = dinv * (a0_ref[...] + hw0_ref[...])
    t1 = dinv * (a1_ref[...] + hw1_ref[...])
    t = jnp.concatenate([t0, t1], axis=1) + bc_ref[...]
    t_ref[...] = t

    @pl.when(pl.program_id(0) == 0)
    def _():
        s1_ref[...] = jnp.zeros_like(s1_ref)
        s2_ref[...] = jnp.zeros_like(s2_ref)

    s1_ref[...] += jnp.sum(t, axis=0, keepdims=True)
    s2_ref[...] += jnp.sum(t * t, axis=0, keepdims=True)


_stats = pl.pallas_call(
    _stats_body,
    grid=(N // RB,),
    in_specs=[
        pl.BlockSpec((RB, HD), lambda i: (i, 0)),
        pl.BlockSpec((RB, HD), lambda i: (i, 0)),
        pl.BlockSpec((RB, HD), lambda i: (i, 0)),
        pl.BlockSpec((RB, HD), lambda i: (i, 0)),
        pl.BlockSpec((RB, 1), lambda i: (i, 0)),
        pl.BlockSpec((1, D), lambda i: (0, 0)),
    ],
    out_specs=[
        pl.BlockSpec((RB, D), lambda i: (i, 0)),
        pl.BlockSpec((1, D), lambda i: (0, 0)),
        pl.BlockSpec((1, D), lambda i: (0, 0)),
    ],
    out_shape=[
        jax.ShapeDtypeStruct((N, D), jnp.float32),
        jax.ShapeDtypeStruct((1, D), jnp.float32),
        jax.ShapeDtypeStruct((1, D), jnp.float32),
    ],
)


def _bn_body(t_ref, s1_ref, s2_ref, g_ref, be_ref, hin_ref, ho_ref):
    inv_n = 1.0 / N
    mu = s1_ref[...] * inv_n
    var = s2_ref[...] * inv_n - mu * mu
    xn = (t_ref[...] - mu) * lax.rsqrt(var + 1e-5) * g_ref[...] + be_ref[...]
    ho_ref[...] = hin_ref[...] + jnp.maximum(xn, 0.0)


_bn_relu_res = pl.pallas_call(
    _bn_body,
    grid=(N // RB,),
    in_specs=[
        pl.BlockSpec((RB, D), lambda i: (i, 0)),
        pl.BlockSpec((1, D), lambda i: (0, 0)),
        pl.BlockSpec((1, D), lambda i: (0, 0)),
        pl.BlockSpec((1, D), lambda i: (0, 0)),
        pl.BlockSpec((1, D), lambda i: (0, 0)),
        pl.BlockSpec((RB, D), lambda i: (i, 0)),
    ],
    out_specs=pl.BlockSpec((RB, D), lambda i: (i, 0)),
    out_shape=jax.ShapeDtypeStruct((N, D), jnp.float32),
)


def _pool_body(h_ref, b_ref, w1_ref, b1_ref, w2_ref, b2_ref, w3_ref, b3_ref,
               out_ref, g_acc, c_acc):
    i = pl.program_id(0)

    @pl.when(i == 0)
    def _():
        g_acc[...] = jnp.zeros_like(g_acc)
        c_acc[...] = jnp.zeros_like(c_acc)

    gid = lax.broadcasted_iota(jnp.int32, (RB, NG), 1)
    onehot = jnp.where(b_ref[...] == gid, 1.0, 0.0).astype(jnp.float32)
    g_acc[...] += lax.dot_general(onehot, h_ref[...], (((0,), (0,)), ((), ())),
                                  preferred_element_type=jnp.float32)
    c_acc[...] += lax.dot_general(onehot, jnp.ones((RB, 1), jnp.float32),
                                  (((0,), (0,)), ((), ())),
                                  preferred_element_type=jnp.float32)

    @pl.when(i == pl.num_programs(0) - 1)
    def _():
        g = g_acc[...] / jnp.maximum(c_acc[...], 1.0)
        g = jnp.maximum(jnp.dot(g, w1_ref[...],
                                preferred_element_type=jnp.float32)
                        + b1_ref[...], 0.0)
        g = jnp.maximum(jnp.dot(g, w2_ref[...],
                                preferred_element_type=jnp.float32)
                        + b2_ref[...], 0.0)
        out_ref[...] = jnp.dot(g, w3_ref[...],
                               preferred_element_type=jnp.float32) + b3_ref[...]


_pool_mlp = pl.pallas_call(
    _pool_body,
    grid=(N // RB,),
    in_specs=[
        pl.BlockSpec((RB, D), lambda i: (i, 0)),
        pl.BlockSpec((RB, 1), lambda i: (i, 0)),
        pl.BlockSpec((D, D // 2), lambda i: (0, 0)),
        pl.BlockSpec((1, D // 2), lambda i: (0, 0)),
        pl.BlockSpec((D // 2, D // 4), lambda i: (0, 0)),
        pl.BlockSpec((1, D // 4), lambda i: (0, 0)),
        pl.BlockSpec((D // 4, 10), lambda i: (0, 0)),
        pl.BlockSpec((1, 10), lambda i: (0, 0)),
    ],
    out_specs=pl.BlockSpec((NG, 10), lambda i: (0, 0)),
    out_shape=jax.ShapeDtypeStruct((NG, 10), jnp.float32),
    scratch_shapes=[
        pltpu.VMEM((NG, D), jnp.float32),
        pltpu.VMEM((NG, 1), jnp.float32),
    ],
)


def kernel(x, edge_index, batch, W0, b0, Wc, bc, gamma, beta,
           W1, b1, W2, b2, W3, b3):
    src = edge_index[0]
    dst = edge_index[1]
    src2d = jnp.concatenate(
        [src, jnp.zeros((EP - E,), jnp.int32)]).reshape(NS * CPT, CH)
    dst2d = jnp.concatenate(
        [dst, jnp.full((EP - E,), DSTPAD, jnp.int32)]).reshape(NS * CPT, CH)
    dstdeg = jnp.concatenate(
        [dst, jnp.full((EPD - E,), DSTPAD, jnp.int32)]).reshape(
            NC * NS * CPW, CH)

    zeros2d = jnp.zeros((640, HD), jnp.float32)
    zeros1d = jnp.zeros((640,), jnp.float32)
    ones1d = jnp.ones((CH,), jnp.float32)

    d0p, d1p = _degree(dstdeg, ones1d, zeros1d)
    d0 = d0p[:N].reshape(N, 1)
    d1 = d1p[:N].reshape(N, 1)

    h, dinv = _embed(x, W0, b0.reshape(1, D), d0, d1)
    for i in range(NL):
        hw0, hw1 = _mm_scale(h, Wc[i], dinv)
        a0, a1 = _edge_scatter(src2d, dst2d, hw0, hw1, zeros2d)
        t, s1, s2 = _stats(a0, a1, hw0, hw1, dinv, bc[i].reshape(1, D))
        h = _bn_relu_res(t, s1, s2, gamma[i].reshape(1, D),
                         beta[i].reshape(1, D), h)

    out = _pool_mlp(h, batch.reshape(N, 1), W1, b1.reshape(1, D // 2),
                    W2, b2.reshape(1, D // 4), W3, b3.reshape(1, 10))
    return out
